# baseline (speedup 1.0000x reference)
import functools

import jax
import jax.numpy as jnp
from jax.experimental import pallas as pl

_TN = 512


def _chamfer_kernel(x1_ref, x2t_ref, d1_ref, d2_ref):
    ni = pl.program_id(1)
    x1 = x1_ref[0]
    x2t = x2t_ref[0]
    dot = jax.lax.dot_general(x1 * -2.0, x2t, (((1,), (0,)), ((), ())),
                              precision=jax.lax.Precision.HIGHEST,
                              preferred_element_type=jnp.float32)
    n1 = jnp.sum(x1 * x1, axis=1, keepdims=True)
    n2 = jnp.sum(x2t * x2t, axis=0, keepdims=True)
    d = (dot + n1) + n2
    d1_ref[...] = jnp.min(d, axis=1).reshape(1, 1, 1, -1)
    m2 = jnp.min(d, axis=0, keepdims=True)

    @pl.when(ni == 0)
    def _init():
        d2_ref[0] = m2

    @pl.when(ni != 0)
    def _acc():
        d2_ref[0] = jnp.minimum(d2_ref[0], m2)


@functools.partial(jax.jit, static_argnames=("interpret",))
def kernel(input1, input2, interpret=False):
    xyz1 = input1 if input1.shape[2] == 3 else jnp.transpose(input1, (0, 2, 1))
    xyz2 = input2 if input2.shape[2] == 3 else jnp.transpose(input2, (0, 2, 1))
    B, N, _ = xyz1.shape
    M = xyz2.shape[1]
    x1p = jnp.pad(xyz1, ((0, 0), (0, 0), (0, 5)))
    x2t = jnp.transpose(jnp.pad(xyz2, ((0, 0), (0, 0), (0, 5))),
                        (0, 2, 1))
    nt = N // _TN
    grid = (B, nt)
    d1, d2 = pl.pallas_call(
        _chamfer_kernel,
        grid=grid,
        in_specs=[
            pl.BlockSpec((1, _TN, 8), lambda b, i: (b, i, 0)),
            pl.BlockSpec((1, 8, M), lambda b, i: (b, 0, 0)),
        ],
        out_specs=[
            pl.BlockSpec((1, 1, 1, _TN), lambda b, i: (b, i, 0, 0)),
            pl.BlockSpec((1, 1, M), lambda b, i: (b, 0, 0)),
        ],
        out_shape=[
            jax.ShapeDtypeStruct((B, nt, 1, _TN), jnp.float32),
            jax.ShapeDtypeStruct((B, 1, M), jnp.float32),
        ],
        interpret=interpret,
    )(x1p, x2t)
    return (d1.reshape(B, N), d2.reshape(B, M))


# speedup vs baseline: 1.5835x; 1.5835x over previous
import functools

import jax
import jax.numpy as jnp
from jax.experimental import pallas as pl

_TN = 2048


def _chamfer_kernel(x1_ref, x2t_ref, d1_ref, d2_ref):
    ni = pl.program_id(1)
    x1 = x1_ref[0]
    x2t = x2t_ref[0]
    dx = x1[:, 0:1] - x2t[0:1, :]
    dy = x1[:, 1:2] - x2t[1:2, :]
    dz = x1[:, 2:3] - x2t[2:3, :]
    d = dx * dx + dy * dy + dz * dz
    d1_ref[...] = jnp.min(d, axis=1).reshape(1, 1, 1, -1)
    m2 = jnp.min(d, axis=0, keepdims=True)

    @pl.when(ni == 0)
    def _init():
        d2_ref[0] = m2

    @pl.when(ni != 0)
    def _acc():
        d2_ref[0] = jnp.minimum(d2_ref[0], m2)


@functools.partial(jax.jit, static_argnames=("interpret",))
def kernel(input1, input2, interpret=False):
    xyz1 = input1 if input1.shape[2] == 3 else jnp.transpose(input1, (0, 2, 1))
    xyz2 = input2 if input2.shape[2] == 3 else jnp.transpose(input2, (0, 2, 1))
    B, N, _ = xyz1.shape
    M = xyz2.shape[1]
    x1p = jnp.pad(xyz1, ((0, 0), (0, 0), (0, 5)))
    x2t = jnp.transpose(jnp.pad(xyz2, ((0, 0), (0, 0), (0, 5))),
                        (0, 2, 1))
    nt = N // _TN
    grid = (B, nt)
    d1, d2 = pl.pallas_call(
        _chamfer_kernel,
        grid=grid,
        in_specs=[
            pl.BlockSpec((1, _TN, 8), lambda b, i: (b, i, 0)),
            pl.BlockSpec((1, 8, M), lambda b, i: (b, 0, 0)),
        ],
        out_specs=[
            pl.BlockSpec((1, 1, 1, _TN), lambda b, i: (b, i, 0, 0)),
            pl.BlockSpec((1, 1, M), lambda b, i: (b, 0, 0)),
        ],
        out_shape=[
            jax.ShapeDtypeStruct((B, nt, 1, _TN), jnp.float32),
            jax.ShapeDtypeStruct((B, 1, M), jnp.float32),
        ],
        interpret=interpret,
    )(x1p, x2t)
    return (d1.reshape(B, N), d2.reshape(B, M))
